# baseline (speedup 1.0000x reference)
import functools

import jax
import jax.numpy as jnp
from jax import lax
from jax.experimental import pallas as pl
from jax.experimental.pallas import tpu as pltpu
from jax.experimental.pallas import tpu_sc as plsc


_N = 4608
_Q = 512
_E = 32
_R = 256
_K = 3
_IMAX = 0x7FFFFFFF

_NC = 2
_NS = 16
_NW = _NC * _NS
_BW = _N // _NW
_L = 16


def _tc_body(x_ref, wt_ref, cand_ref):
    xb = x_ref[...]
    wt = wt_ref[...]

    wn = jnp.sum(wt * wt, axis=0)[None, :]
    xw2 = jax.lax.dot(xb, wt + wt,
                      precision=jax.lax.Precision.HIGHEST)
    s = wn - xw2

    u = jax.lax.bitcast_convert_type(s, jnp.int32)
    k = u ^ jax.lax.shift_right_logical(
        jax.lax.shift_right_arithmetic(u, 31), 1)
    qiota = jax.lax.broadcasted_iota(jnp.int32, (_R, _Q), 1)
    key = (k & jnp.int32(~511)) | qiota

    for kk in range(_K):
        mk = jnp.min(key, axis=1, keepdims=True)
        key = jnp.where(key == mk, _IMAX, key)
        cand_ref[0, kk, :] = mk[:, 0] & 511


def _sc_phase2(w_hbm, x_hbm, cand_hbm, qd_hbm, qi_hbm,
               w_v, x_v, cand_v, out_v, qi_v):
    wid = lax.axis_index("s") * _NC + lax.axis_index("c")
    base = wid * _BW

    pltpu.sync_copy(w_hbm, w_v)
    pltpu.sync_copy(x_hbm.at[pl.ds(base * _E, _BW * _E)], x_v)
    for kk in range(_K):
        pltpu.sync_copy(cand_hbm.at[pl.ds(kk * _N + base, _BW)],
                        cand_v.at[pl.ds(kk * _BW, _BW)])

    lanes = lax.iota(jnp.int32, _L)
    for g in range(_BW // _L):
        xbase = (lanes + g * _L) * _E
        xe = [plsc.load_gather(x_v, [xbase + e]) for e in range(_E)]

        best_d = None
        best_i = None
        for kk in range(_K):
            idx = cand_v[pl.ds(kk * _BW + g * _L, _L)]
            wbase = idx * _E
            blocks = []
            for blk in range(4):
                sq = []
                for e in range(8):
                    ee = 8 * blk + e
                    wv = plsc.load_gather(w_v, [wbase + ee])
                    d = wv - xe[ee]
                    sq.append(d * d)
                t0 = [sq[i] + sq[i + 4] for i in range(4)]
                t1 = [t0[0] + t0[2], t0[1] + t0[3]]
                blocks.append(t1[0] + t1[1])
            dist = ((blocks[0] + blocks[1]) + blocks[2]) + blocks[3]
            if best_d is None:
                best_d, best_i = dist, idx
            else:
                take = (dist < best_d) | ((dist == best_d) & (idx < best_i))
                best_d = jnp.where(take, dist, best_d)
                best_i = jnp.where(take, idx, best_i)

        bbase = best_i * _E
        for e in range(_E):
            wv = plsc.load_gather(w_v, [bbase + e])
            plsc.store_scatter(out_v, [xbase + e], wv)
        qi_v[pl.ds(g * _L, _L)] = best_i

    pltpu.sync_copy(out_v, qd_hbm.at[pl.ds(base * _E, _BW * _E)])
    pltpu.sync_copy(qi_v, qi_hbm.at[pl.ds(base, _BW)])


@jax.jit
def _vq(xf, wt, wf):
    nb = _N // _R
    cand = pl.pallas_call(
        _tc_body,
        grid=(nb,),
        in_specs=[
            pl.BlockSpec((_R, _E), lambda i: (i, 0)),
            pl.BlockSpec((_E, _Q), lambda i: (0, 0)),
        ],
        out_specs=pl.BlockSpec((1, _K, _R), lambda i: (i, 0, 0)),
        out_shape=jax.ShapeDtypeStruct((nb, _K, _R), jnp.int32),
    )(xf, wt)
    cand_flat = jnp.transpose(cand, (1, 0, 2)).reshape(_K * _N)
    sc_call = pl.kernel(
        _sc_phase2,
        mesh=plsc.VectorSubcoreMesh(core_axis_name="c", subcore_axis_name="s"),
        compiler_params=pltpu.CompilerParams(needs_layout_passes=False),
        out_type=[
            jax.ShapeDtypeStruct((_N * _E,), jnp.float32),
            jax.ShapeDtypeStruct((_N,), jnp.int32),
        ],
        scratch_types=[
            pltpu.VMEM((_Q * _E,), jnp.float32),
            pltpu.VMEM((_BW * _E,), jnp.float32),
            pltpu.VMEM((_K * _BW,), jnp.int32),
            pltpu.VMEM((_BW * _E,), jnp.float32),
            pltpu.VMEM((_BW,), jnp.int32),
        ],
    )
    qd_flat, qi = sc_call(wf, xf.reshape(_N * _E), cand_flat)
    return qd_flat, qi


def kernel(x, weight):
    xf = x.reshape(_N, _E)
    qd_flat, qi = _vq(xf, weight.T, weight.reshape(_Q * _E))
    return qd_flat.reshape(x.shape), qi.reshape(x.shape[:-1])


# speedup vs baseline: 1.2255x; 1.2255x over previous
import functools

import jax
import jax.numpy as jnp
from jax.experimental import pallas as pl


_N = 4608
_Q = 512
_E = 32
_R = 128


def _vq_body(x_ref, wt_ref, w_ref, qd_ref, qi_ref):
    xb = x_ref[...]
    wt = wt_ref[...]

    block_sums = []
    for g in range(4):
        sq = []
        for e in range(8):
            ee = 8 * g + e
            d = wt[ee, :][None, :] - xb[:, ee][:, None]
            sq.append(d * d)
        t0 = [sq[i] + sq[i + 4] for i in range(4)]
        t1 = [t0[0] + t0[2], t0[1] + t0[3]]
        block_sums.append(t1[0] + t1[1])
    dist = ((block_sums[0] + block_sums[1]) + block_sums[2]) + block_sums[3]

    m = jnp.min(dist, axis=1, keepdims=True)
    qiota = jax.lax.broadcasted_iota(jnp.int32, (_R, _Q), 1)
    idx = jnp.min(jnp.where(dist == m, qiota, _Q), axis=1)

    onehot = (qiota == idx[:, None]).astype(jnp.float32)
    qd_ref[...] = jax.lax.dot(onehot, w_ref[...],
                              precision=jax.lax.Precision.HIGHEST)
    qi_ref[0, 0, :] = idx


@jax.jit
def _vq(xf, wt, w):
    nb = _N // _R
    qd, qi = pl.pallas_call(
        _vq_body,
        grid=(nb,),
        in_specs=[
            pl.BlockSpec((_R, _E), lambda i: (i, 0)),
            pl.BlockSpec((_E, _Q), lambda i: (0, 0)),
            pl.BlockSpec((_Q, _E), lambda i: (0, 0)),
        ],
        out_specs=[
            pl.BlockSpec((_R, _E), lambda i: (i, 0)),
            pl.BlockSpec((1, 1, _R), lambda i: (i, 0, 0)),
        ],
        out_shape=[
            jax.ShapeDtypeStruct((_N, _E), jnp.float32),
            jax.ShapeDtypeStruct((nb, 1, _R), jnp.int32),
        ],
    )(xf, wt, w)
    return qd, qi


def kernel(x, weight):
    xf = x.reshape(_N, _E)
    qd, qi = _vq(xf, weight.T, weight)
    return qd.reshape(x.shape), qi.reshape(x.shape[:-1])


# speedup vs baseline: 1.2763x; 1.0414x over previous
import functools

import jax
import jax.numpy as jnp
from jax.experimental import pallas as pl


_N = 4608
_Q = 512
_E = 32
_R = 576


def _vq_body(x_ref, wt_ref, w_ref, qd_ref, qi_ref):
    xb = x_ref[...]
    wt = wt_ref[...]

    block_sums = []
    for g in range(4):
        sq = []
        for e in range(8):
            ee = 8 * g + e
            d = wt[ee, :][None, :] - xb[:, ee][:, None]
            sq.append(d * d)
        t0 = [sq[i] + sq[i + 4] for i in range(4)]
        t1 = [t0[0] + t0[2], t0[1] + t0[3]]
        block_sums.append(t1[0] + t1[1])
    dist = ((block_sums[0] + block_sums[1]) + block_sums[2]) + block_sums[3]

    m = jnp.min(dist, axis=1, keepdims=True)
    qiota = jax.lax.broadcasted_iota(jnp.int32, (_R, _Q), 1)
    idx = jnp.min(jnp.where(dist == m, qiota, _Q), axis=1)

    onehot = (qiota == idx[:, None]).astype(jnp.float32)
    qd_ref[...] = jax.lax.dot(onehot, w_ref[...],
                              precision=jax.lax.Precision.HIGHEST)
    qi_ref[0, 0, :] = idx


@jax.jit
def _vq(xf, wt, w):
    nb = _N // _R
    qd, qi = pl.pallas_call(
        _vq_body,
        grid=(nb,),
        in_specs=[
            pl.BlockSpec((_R, _E), lambda i: (i, 0)),
            pl.BlockSpec((_E, _Q), lambda i: (0, 0)),
            pl.BlockSpec((_Q, _E), lambda i: (0, 0)),
        ],
        out_specs=[
            pl.BlockSpec((_R, _E), lambda i: (i, 0)),
            pl.BlockSpec((1, 1, _R), lambda i: (i, 0, 0)),
        ],
        out_shape=[
            jax.ShapeDtypeStruct((_N, _E), jnp.float32),
            jax.ShapeDtypeStruct((nb, 1, _R), jnp.int32),
        ],
    )(xf, wt, w)
    return qd, qi


def kernel(x, weight):
    xf = x.reshape(_N, _E)
    qd, qi = _vq(xf, weight.T, weight)
    return qd.reshape(x.shape), qi.reshape(x.shape[:-1])


# speedup vs baseline: 1.4705x; 1.1522x over previous
import jax
import jax.numpy as jnp
from jax.experimental import pallas as pl


_N = 4608
_Q = 512
_E = 32
_R = 256
_K = 3
_IMAX = 0x7FFFFFFF


def _exact_dist(wrow, xb):
    d = wrow - xb
    sq = d * d
    blocks = []
    for g in range(4):
        b = sq[:, 8 * g:8 * g + 8]
        u = b[:, 0:4] + b[:, 4:8]
        v = u[:, 0:2] + u[:, 2:4]
        blocks.append(v[:, 0:1] + v[:, 1:2])
    return ((blocks[0] + blocks[1]) + blocks[2]) + blocks[3]


def _vq_body(x_ref, wt_ref, whi_ref, wlo_ref, wll_ref, qd_ref, qi_ref):
    xb = x_ref[...]
    wt = wt_ref[...]

    wn = jnp.sum(wt * wt, axis=0)[None, :]
    xw2 = jax.lax.dot(xb, wt + wt,
                      precision=jax.lax.Precision.HIGHEST)
    s = wn - xw2

    u = jax.lax.bitcast_convert_type(s, jnp.int32)
    k = u ^ jax.lax.shift_right_logical(
        jax.lax.shift_right_arithmetic(u, 31), 1)
    qiota = jax.lax.broadcasted_iota(jnp.int32, (_R, _Q), 1)
    key = (k & jnp.int32(~511)) | qiota

    hits = []
    idxs = []
    for _ in range(_K):
        mk = jnp.min(key, axis=1, keepdims=True)
        hit = key == mk
        key = jnp.where(hit, _IMAX, key)
        hits.append(hit)
        idxs.append(mk[:, 0] & 511)

    onehot = jnp.concatenate(hits, axis=0).astype(jnp.bfloat16)
    rows = (
        jax.lax.dot(onehot, whi_ref[...],
                    preferred_element_type=jnp.float32)
        + jax.lax.dot(onehot, wlo_ref[...],
                      preferred_element_type=jnp.float32)
        + jax.lax.dot(onehot, wll_ref[...],
                      preferred_element_type=jnp.float32)
    )
    xb3 = jnp.concatenate([xb] * _K, axis=0)
    dall = _exact_dist(rows, xb3)[:, 0]

    best_d = dall[0:_R]
    best_i = idxs[0]
    best_row = rows[0:_R, :]
    for kk in range(1, _K):
        d = dall[kk * _R:(kk + 1) * _R]
        idx = idxs[kk]
        take = (d < best_d) | ((d == best_d) & (idx < best_i))
        best_d = jnp.where(take, d, best_d)
        best_i = jnp.where(take, idx, best_i)
        best_row = jnp.where(take[:, None], rows[kk * _R:(kk + 1) * _R, :],
                             best_row)

    qd_ref[...] = best_row
    qi_ref[0, 0, :] = best_i


@jax.jit
def _vq(xf, wt, whi, wlo, wll):
    nb = _N // _R
    qd, qi = pl.pallas_call(
        _vq_body,
        grid=(nb,),
        in_specs=[
            pl.BlockSpec((_R, _E), lambda i: (i, 0)),
            pl.BlockSpec((_E, _Q), lambda i: (0, 0)),
            pl.BlockSpec((_Q, _E), lambda i: (0, 0)),
            pl.BlockSpec((_Q, _E), lambda i: (0, 0)),
            pl.BlockSpec((_Q, _E), lambda i: (0, 0)),
        ],
        out_specs=[
            pl.BlockSpec((_R, _E), lambda i: (i, 0)),
            pl.BlockSpec((1, 1, _R), lambda i: (i, 0, 0)),
        ],
        out_shape=[
            jax.ShapeDtypeStruct((_N, _E), jnp.float32),
            jax.ShapeDtypeStruct((nb, 1, _R), jnp.int32),
        ],
    )(xf, wt, whi, wlo, wll)
    return qd, qi


def kernel(x, weight):
    xf = x.reshape(_N, _E)
    whi = weight.astype(jnp.bfloat16)
    r1 = weight - whi.astype(jnp.float32)
    wlo = r1.astype(jnp.bfloat16)
    wll = (r1 - wlo.astype(jnp.float32)).astype(jnp.bfloat16)
    qd, qi = _vq(xf, weight.T, whi, wlo, wll)
    return qd.reshape(x.shape), qi.reshape(x.shape[:-1])
